# baseline (speedup 1.0000x reference)
import functools

import jax
import jax.numpy as jnp
from jax import lax
from jax.experimental import pallas as pl
from jax.experimental.pallas import tpu as pltpu
from jax.experimental.pallas import tpu_sc as plsc

N = 10000
D = 128
E = 160000
NSUB = 16
NCORE = 2
NP = 10240
C = 128
CPT = 80
NCH_R = NSUB * CPT
EP = NCH_R * C
RPT = NP // NSUB
NSTG = 5
SPC = CPT // NSTG


def _regcopy_row(dst_ref, src_ref, j):
    for k in range(C // 16):
        dst_ref[pl.ds(k * 16, 16)] = src_ref[j, pl.ds(k * 16, 16)]


def _sc_mesh():
    return plsc.VectorSubcoreMesh(core_axis_name="c", subcore_axis_name="s",
                                  num_cores=NCORE, num_subcores=NSUB)


def _deg_body(srcs_hbm, dsts_hbm, zflat_hbm, out_hbm,
              srcb_v, dstb_v, scur_v, dcur_v, ones_v, dsrc_s, ddst_s):
    cid = lax.axis_index("c")
    sid = lax.axis_index("s")
    rpt2 = (2 * NP) // NSUB
    base = cid * NCH_R + sid * CPT
    pltpu.sync_copy(srcs_hbm.at[pl.ds(base, CPT)], srcb_v)
    pltpu.sync_copy(dsts_hbm.at[pl.ds(base, CPT)], dstb_v)
    pltpu.sync_copy(zflat_hbm.at[pl.ds(sid * rpt2, rpt2)],
                    dsrc_s.at[pl.ds(sid * rpt2, rpt2)])
    pltpu.sync_copy(zflat_hbm.at[pl.ds(sid * RPT, RPT)],
                    ddst_s.at[pl.ds(sid * RPT, RPT)])
    for k in range(C // 16):
        ones_v[pl.ds(k * 16, 16)] = jnp.ones((16,), jnp.float32)
    plsc.subcore_barrier()

    def body(j, carry):
        _regcopy_row(scur_v, srcb_v, j)
        _regcopy_row(dcur_v, dstb_v, j)
        pltpu.sync_copy(ones_v, dsrc_s.at[scur_v], add=True)
        pltpu.sync_copy(ones_v, ddst_s.at[dcur_v], add=True)
        return carry

    lax.fori_loop(0, CPT, body, 0)
    plsc.subcore_barrier()
    pltpu.sync_copy(dsrc_s.at[pl.ds(cid * NP + sid * RPT, RPT)],
                    out_hbm.at[pl.ds((2 * cid) * NP + sid * RPT, RPT)])
    pltpu.sync_copy(ddst_s.at[pl.ds(sid * RPT, RPT)],
                    out_hbm.at[pl.ds((2 * cid + 1) * NP + sid * RPT, RPT)])


def _spmm_body(xs_hbm, srcs_hbm, dsts_hbm, zrows_hbm, g_hbm,
               srcb_v, dstb_v, dcur0_v, rows0_v, acc_s, sem0):
    cid = lax.axis_index("c")
    sid = lax.axis_index("s")
    base = cid * NCH_R + sid * CPT
    pltpu.sync_copy(zrows_hbm.at[pl.ds(sid * RPT, RPT)],
                    acc_s.at[pl.ds(sid * RPT, RPT)])
    plsc.subcore_barrier()

    nbig = SPC // 2
    for s in range(NSTG):
        sbase = (base + s * SPC) * C
        pltpu.sync_copy(srcs_hbm.at[pl.ds(sbase, SPC * C)], srcb_v)
        pltpu.sync_copy(dsts_hbm.at[pl.ds(sbase, SPC * C)], dstb_v)

        def body(i, carry):
            a = 2 * i
            pltpu.async_copy(xs_hbm.at[srcb_v.at[pl.ds(a * C, 2 * C)]],
                             rows0_v, sem0)
            for k in range(2 * C // 16):
                dcur0_v[pl.ds(k * 16, 16)] = dstb_v[pl.ds(a * C + k * 16, 16)]
            pltpu.make_async_copy(xs_hbm, rows0_v, sem0).wait()
            pltpu.sync_copy(rows0_v, acc_s.at[dcur0_v], add=True)
            return carry

        lax.fori_loop(0, nbig, body, 0)
    plsc.subcore_barrier()
    pltpu.sync_copy(acc_s.at[pl.ds(sid * RPT, RPT)],
                    g_hbm.at[pl.ds(cid * NP + sid * RPT, RPT)])


def _deg_call():
    return pl.kernel(
        _deg_body,
        out_type=jax.ShapeDtypeStruct((4 * NP,), jnp.float32),
        mesh=_sc_mesh(),
        scratch_types=[
            pltpu.VMEM((CPT, C), jnp.int32),
            pltpu.VMEM((CPT, C), jnp.int32),
            pltpu.VMEM((C,), jnp.int32),
            pltpu.VMEM((C,), jnp.int32),
            pltpu.VMEM((C,), jnp.float32),
            pltpu.VMEM_SHARED((2 * NP,), jnp.float32),
            pltpu.VMEM_SHARED((NP,), jnp.float32),
        ],
    )


def _spmm_call():
    return pl.kernel(
        _spmm_body,
        out_type=jax.ShapeDtypeStruct((2 * NP, D), jnp.float32),
        mesh=_sc_mesh(),
        scratch_types=[
            pltpu.VMEM((SPC * C,), jnp.int32),
            pltpu.VMEM((SPC * C,), jnp.int32),
            pltpu.VMEM((2 * C,), jnp.int32),
            pltpu.VMEM((2 * C, D), jnp.float32),
            pltpu.VMEM_SHARED((NP, D), jnp.float32),
            pltpu.SemaphoreType.DMA,
        ],
    )


def _prep_body(xp_ref, degc_ref, xs_ref, csd_ref):
    iot = lax.broadcasted_iota(jnp.int32, (NP, 1), 0)
    dg = degc_ref[...]
    valid = (dg > 0) & (iot < N)
    csd = jnp.where(valid, lax.rsqrt(jnp.maximum(dg, 1.0)), 0.0)
    csd_ref[...] = csd
    xpv = xp_ref[...]
    xs_ref[0:NP, :] = xpv * csd[:, 0:1]
    xs_ref[NP:2 * NP, :] = xpv * csd[:, 2:3]


def _dense_mid_body(g_ref, csd_ref, w_ref, b_ref, xsn_ref):
    a0 = g_ref[0:N, :] * csd_ref[0:N, 1:2]
    a1 = g_ref[NP:NP + N, :] * csd_ref[0:N, 3:4]
    t = (jnp.dot(a0, w_ref[0], preferred_element_type=jnp.float32)
         + jnp.dot(a1, w_ref[1], preferred_element_type=jnp.float32)
         + b_ref[0:1, :] + b_ref[1:2, :])
    h = jnp.where(t > 0, t, 0.25 * t)
    xsn_ref[0:N, :] = h * csd_ref[0:N, 0:1]
    xsn_ref[N:NP, :] = jnp.zeros((NP - N, D), jnp.float32)
    xsn_ref[NP:NP + N, :] = h * csd_ref[0:N, 2:3]
    xsn_ref[NP + N:2 * NP, :] = jnp.zeros((NP - N, D), jnp.float32)


def _dense_last_body(g_ref, csd_ref, w_ref, b_ref, h_ref):
    a0 = g_ref[0:N, :] * csd_ref[0:N, 1:2]
    a1 = g_ref[NP:NP + N, :] * csd_ref[0:N, 3:4]
    t = (jnp.dot(a0, w_ref[0], preferred_element_type=jnp.float32)
         + jnp.dot(a1, w_ref[1], preferred_element_type=jnp.float32)
         + b_ref[0:1, :] + b_ref[1:2, :])
    h_ref[...] = jnp.where(t > 0, t, 0.25 * t)


_prep_call = pl.pallas_call(
    _prep_body,
    out_shape=(jax.ShapeDtypeStruct((2 * NP, D), jnp.float32),
               jax.ShapeDtypeStruct((NP, 4), jnp.float32)),
)

_dense_mid_call = pl.pallas_call(
    _dense_mid_body,
    out_shape=jax.ShapeDtypeStruct((2 * NP, D), jnp.float32),
)

_dense_last_call = pl.pallas_call(
    _dense_last_body,
    out_shape=jax.ShapeDtypeStruct((N, D), jnp.float32),
)


def kernel(x, edge_index_r0, edge_index_r1, W1, b1, W2, b2, W3, b3):
    srcs = jnp.stack([edge_index_r0[0], edge_index_r1[0]]).astype(jnp.int32)
    dsts = jnp.stack([edge_index_r0[1], edge_index_r1[1]]).astype(jnp.int32)
    srcs_p = jnp.pad(srcs, ((0, 0), (0, EP - E)), constant_values=N)
    dsts_p = jnp.pad(dsts, ((0, 0), (0, EP - E)), constant_values=N)
    srcs_g = (srcs_p + jnp.array([[0], [NP]], jnp.int32)).reshape(2 * NCH_R, C)
    dsts_g = dsts_p.reshape(2 * NCH_R, C)

    zflat = jnp.zeros((2 * NP,), jnp.float32)
    zrows = jnp.zeros((NP, D), jnp.float32)
    xp = jnp.pad(x.astype(jnp.float32), ((0, NP - N), (0, 0)))

    degs_flat = _deg_call()(srcs_g, dsts_g, zflat)
    degc = degs_flat.reshape(4, NP).T
    xs, csd = _prep_call(xp, degc)

    for li, (W, b) in enumerate(((W1, b1), (W2, b2), (W3, b3))):
        g = _spmm_call()(xs, srcs_g.reshape(-1), dsts_g.reshape(-1), zrows)
        if li < 2:
            xs = _dense_mid_call(g, csd, W, b)
        else:
            return _dense_last_call(g, csd, W, b)


# speedup vs baseline: 1.1105x; 1.1105x over previous
import functools

import jax
import jax.numpy as jnp
from jax import lax
from jax.experimental import pallas as pl
from jax.experimental.pallas import tpu as pltpu
from jax.experimental.pallas import tpu_sc as plsc

N = 10000
D = 128
E = 160000
NSUB = 16
NCORE = 2
NP = 10240
C = 128
CPT = 80
NCH_R = NSUB * CPT
EP = NCH_R * C
RPT = NP // NSUB
NSTG = 5
SPC = CPT // NSTG
EPB = 160
SPB = 2560


def _regcopy_row(dst_ref, src_ref, j):
    for k in range(C // 16):
        dst_ref[pl.ds(k * 16, 16)] = src_ref[j, pl.ds(k * 16, 16)]


def _sc_mesh():
    return plsc.VectorSubcoreMesh(core_axis_name="c", subcore_axis_name="s",
                                  num_cores=NCORE, num_subcores=NSUB)


def _deg_body(srcs_hbm, dsts_hbm, zflat_hbm, out_hbm,
              srcb_v, dstb_v, scur_v, dcur_v, ones_v, dsrc_s, ddst_s):
    cid = lax.axis_index("c")
    sid = lax.axis_index("s")
    rpt2 = (2 * NP) // NSUB
    base = cid * NCH_R + sid * CPT
    pltpu.sync_copy(srcs_hbm.at[pl.ds(base, CPT)], srcb_v)
    pltpu.sync_copy(dsts_hbm.at[pl.ds(base, CPT)], dstb_v)
    pltpu.sync_copy(zflat_hbm.at[pl.ds(sid * rpt2, rpt2)],
                    dsrc_s.at[pl.ds(sid * rpt2, rpt2)])
    pltpu.sync_copy(zflat_hbm.at[pl.ds(sid * RPT, RPT)],
                    ddst_s.at[pl.ds(sid * RPT, RPT)])
    for k in range(C // 16):
        ones_v[pl.ds(k * 16, 16)] = jnp.ones((16,), jnp.float32)
    plsc.subcore_barrier()

    def body(j, carry):
        _regcopy_row(scur_v, srcb_v, j)
        _regcopy_row(dcur_v, dstb_v, j)
        pltpu.sync_copy(ones_v, dsrc_s.at[scur_v], add=True)
        pltpu.sync_copy(ones_v, ddst_s.at[dcur_v], add=True)
        return carry

    lax.fori_loop(0, CPT, body, 0)
    plsc.subcore_barrier()
    pltpu.sync_copy(dsrc_s.at[pl.ds(cid * NP + sid * RPT, RPT)],
                    out_hbm.at[pl.ds((2 * cid) * NP + sid * RPT, RPT)])
    pltpu.sync_copy(ddst_s.at[pl.ds(sid * RPT, RPT)],
                    out_hbm.at[pl.ds((2 * cid + 1) * NP + sid * RPT, RPT)])


def _spmm_body(xs_hbm, srcs_hbm, dsts_hbm, zrows_hbm, g_hbm,
               srcb_v, dstb_v, dcur0_v, dcur1_v, rows0_v, rows1_v, acc_s,
               sem0, sem1):
    cid = lax.axis_index("c")
    sid = lax.axis_index("s")
    base = cid * NCH_R + sid * CPT
    pltpu.sync_copy(zrows_hbm.at[pl.ds(sid * RPT, RPT)],
                    acc_s.at[pl.ds(sid * RPT, RPT)])
    plsc.subcore_barrier()

    def _dcur(dcur_v, a):
        for k in range(EPB // 16):
            dcur_v[pl.ds(k * 16, 16)] = dstb_v[pl.ds(a * EPB + k * 16, 16)]

    npair = SPB // EPB // 2
    for s in range(CPT * C // SPB):
        sbase = base * C + s * SPB
        pltpu.sync_copy(srcs_hbm.at[pl.ds(sbase, SPB)], srcb_v)
        pltpu.sync_copy(dsts_hbm.at[pl.ds(sbase, SPB)], dstb_v)
        pltpu.async_copy(xs_hbm.at[srcb_v.at[pl.ds(0, EPB)]], rows0_v, sem0)
        _dcur(dcur0_v, 0)

        def body(i, carry):
            a = 2 * i
            pltpu.async_copy(xs_hbm.at[srcb_v.at[pl.ds((a + 1) * EPB, EPB)]],
                             rows1_v, sem1)
            _dcur(dcur1_v, a + 1)
            pltpu.make_async_copy(xs_hbm, rows0_v, sem0).wait()
            pltpu.sync_copy(rows0_v, acc_s.at[dcur0_v], add=True)

            @pl.when(i < npair - 1)
            def _():
                pltpu.async_copy(
                    xs_hbm.at[srcb_v.at[pl.ds((a + 2) * EPB, EPB)]],
                    rows0_v, sem0)
                _dcur(dcur0_v, a + 2)

            pltpu.make_async_copy(xs_hbm, rows1_v, sem1).wait()
            pltpu.sync_copy(rows1_v, acc_s.at[dcur1_v], add=True)
            return carry

        lax.fori_loop(0, npair, body, 0)
    plsc.subcore_barrier()
    pltpu.sync_copy(acc_s.at[pl.ds(sid * RPT, RPT)],
                    g_hbm.at[pl.ds(cid * NP + sid * RPT, RPT)])


def _deg_call():
    return pl.kernel(
        _deg_body,
        out_type=jax.ShapeDtypeStruct((4 * NP,), jnp.float32),
        mesh=_sc_mesh(),
        scratch_types=[
            pltpu.VMEM((CPT, C), jnp.int32),
            pltpu.VMEM((CPT, C), jnp.int32),
            pltpu.VMEM((C,), jnp.int32),
            pltpu.VMEM((C,), jnp.int32),
            pltpu.VMEM((C,), jnp.float32),
            pltpu.VMEM_SHARED((2 * NP,), jnp.float32),
            pltpu.VMEM_SHARED((NP,), jnp.float32),
        ],
    )


def _spmm_call():
    return pl.kernel(
        _spmm_body,
        out_type=jax.ShapeDtypeStruct((2 * NP, D), jnp.float32),
        mesh=_sc_mesh(),
        scratch_types=[
            pltpu.VMEM((SPB,), jnp.int32),
            pltpu.VMEM((SPB,), jnp.int32),
            pltpu.VMEM((EPB,), jnp.int32),
            pltpu.VMEM((EPB,), jnp.int32),
            pltpu.VMEM((EPB, D), jnp.float32),
            pltpu.VMEM((EPB, D), jnp.float32),
            pltpu.VMEM_SHARED((NP, D), jnp.float32),
            pltpu.SemaphoreType.DMA,
            pltpu.SemaphoreType.DMA,
        ],
    )


def _prep_body(xp_ref, degc_ref, xs_ref, csd_ref):
    iot = lax.broadcasted_iota(jnp.int32, (NP, 1), 0)
    dg = degc_ref[...]
    valid = (dg > 0) & (iot < N)
    csd = jnp.where(valid, lax.rsqrt(jnp.maximum(dg, 1.0)), 0.0)
    csd_ref[...] = csd
    xpv = xp_ref[...]
    xs_ref[0:NP, :] = xpv * csd[:, 0:1]
    xs_ref[NP:2 * NP, :] = xpv * csd[:, 2:3]


def _dense_mid_body(g_ref, csd_ref, w_ref, b_ref, xsn_ref):
    a0 = g_ref[0:N, :] * csd_ref[0:N, 1:2]
    a1 = g_ref[NP:NP + N, :] * csd_ref[0:N, 3:4]
    t = (jnp.dot(a0, w_ref[0], preferred_element_type=jnp.float32)
         + jnp.dot(a1, w_ref[1], preferred_element_type=jnp.float32)
         + b_ref[0:1, :] + b_ref[1:2, :])
    h = jnp.where(t > 0, t, 0.25 * t)
    xsn_ref[0:N, :] = h * csd_ref[0:N, 0:1]
    xsn_ref[N:NP, :] = jnp.zeros((NP - N, D), jnp.float32)
    xsn_ref[NP:NP + N, :] = h * csd_ref[0:N, 2:3]
    xsn_ref[NP + N:2 * NP, :] = jnp.zeros((NP - N, D), jnp.float32)


def _dense_last_body(g_ref, csd_ref, w_ref, b_ref, h_ref):
    a0 = g_ref[0:N, :] * csd_ref[0:N, 1:2]
    a1 = g_ref[NP:NP + N, :] * csd_ref[0:N, 3:4]
    t = (jnp.dot(a0, w_ref[0], preferred_element_type=jnp.float32)
         + jnp.dot(a1, w_ref[1], preferred_element_type=jnp.float32)
         + b_ref[0:1, :] + b_ref[1:2, :])
    h_ref[...] = jnp.where(t > 0, t, 0.25 * t)


_prep_call = pl.pallas_call(
    _prep_body,
    out_shape=(jax.ShapeDtypeStruct((2 * NP, D), jnp.float32),
               jax.ShapeDtypeStruct((NP, 4), jnp.float32)),
)

_dense_mid_call = pl.pallas_call(
    _dense_mid_body,
    out_shape=jax.ShapeDtypeStruct((2 * NP, D), jnp.float32),
)

_dense_last_call = pl.pallas_call(
    _dense_last_body,
    out_shape=jax.ShapeDtypeStruct((N, D), jnp.float32),
)


def kernel(x, edge_index_r0, edge_index_r1, W1, b1, W2, b2, W3, b3):
    srcs = jnp.stack([edge_index_r0[0], edge_index_r1[0]]).astype(jnp.int32)
    dsts = jnp.stack([edge_index_r0[1], edge_index_r1[1]]).astype(jnp.int32)
    srcs_p = jnp.pad(srcs, ((0, 0), (0, EP - E)), constant_values=N)
    dsts_p = jnp.pad(dsts, ((0, 0), (0, EP - E)), constant_values=N)
    srcs_g = (srcs_p + jnp.array([[0], [NP]], jnp.int32)).reshape(2 * NCH_R, C)
    dsts_g = dsts_p.reshape(2 * NCH_R, C)

    zflat = jnp.zeros((2 * NP,), jnp.float32)
    zrows = jnp.zeros((NP, D), jnp.float32)
    xp = jnp.pad(x.astype(jnp.float32), ((0, NP - N), (0, 0)))

    degs_flat = _deg_call()(srcs_g, dsts_g, zflat)
    degc = degs_flat.reshape(4, NP).T
    xs, csd = _prep_call(xp, degc)

    for li, (W, b) in enumerate(((W1, b1), (W2, b2), (W3, b3))):
        g = _spmm_call()(xs, srcs_g.reshape(-1), dsts_g.reshape(-1), zrows)
        if li < 2:
            xs = _dense_mid_call(g, csd, W, b)
        else:
            return _dense_last_call(g, csd, W, b)


# speedup vs baseline: 1.3131x; 1.1824x over previous
import functools

import jax
import jax.numpy as jnp
from jax import lax
from jax.experimental import pallas as pl
from jax.experimental.pallas import tpu as pltpu
from jax.experimental.pallas import tpu_sc as plsc

N = 10000
D = 128
E = 160000
NSUB = 16
NCORE = 2
NP = 10240
C = 128
CPT = 80
NCH_R = NSUB * CPT
EP = NCH_R * C
RPT = NP // NSUB
NSTG = 2
SPC = CPT // NSTG


def _regcopy_row(dst_ref, src_ref, j):
    for k in range(C // 16):
        dst_ref[pl.ds(k * 16, 16)] = src_ref[j, pl.ds(k * 16, 16)]


def _sc_mesh():
    return plsc.VectorSubcoreMesh(core_axis_name="c", subcore_axis_name="s",
                                  num_cores=NCORE, num_subcores=NSUB)


def _deg_body(srcs_hbm, dsts_hbm, zflat_hbm, out_hbm,
              srcb_v, dstb_v, scur_v, dcur_v, ones_v, dsrc_s, ddst_s):
    cid = lax.axis_index("c")
    sid = lax.axis_index("s")
    rpt2 = (2 * NP) // NSUB
    base = cid * NCH_R + sid * CPT
    pltpu.sync_copy(srcs_hbm.at[pl.ds(base, CPT)], srcb_v)
    pltpu.sync_copy(dsts_hbm.at[pl.ds(base, CPT)], dstb_v)
    pltpu.sync_copy(zflat_hbm.at[pl.ds(sid * rpt2, rpt2)],
                    dsrc_s.at[pl.ds(sid * rpt2, rpt2)])
    pltpu.sync_copy(zflat_hbm.at[pl.ds(sid * RPT, RPT)],
                    ddst_s.at[pl.ds(sid * RPT, RPT)])
    for k in range(C // 16):
        ones_v[pl.ds(k * 16, 16)] = jnp.ones((16,), jnp.float32)
    plsc.subcore_barrier()

    def body(j, carry):
        _regcopy_row(scur_v, srcb_v, j)
        _regcopy_row(dcur_v, dstb_v, j)
        pltpu.sync_copy(ones_v, dsrc_s.at[scur_v], add=True)
        pltpu.sync_copy(ones_v, ddst_s.at[dcur_v], add=True)
        return carry

    lax.fori_loop(0, CPT, body, 0)
    plsc.subcore_barrier()
    pltpu.sync_copy(dsrc_s.at[pl.ds(cid * NP + sid * RPT, RPT)],
                    out_hbm.at[pl.ds((2 * cid) * NP + sid * RPT, RPT)])
    pltpu.sync_copy(ddst_s.at[pl.ds(sid * RPT, RPT)],
                    out_hbm.at[pl.ds((2 * cid + 1) * NP + sid * RPT, RPT)])


def _spmm_body(xs_hbm, srcs_hbm, dsts_hbm, zrows_hbm, g_hbm,
               srcb_v, dstb_v, dcur0_v, dcur1_v, rows0_v, rows1_v, acc_s,
               sem0, sem1):
    cid = lax.axis_index("c")
    sid = lax.axis_index("s")
    base = cid * NCH_R + sid * CPT
    pltpu.sync_copy(zrows_hbm.at[pl.ds(sid * RPT, RPT)],
                    acc_s.at[pl.ds(sid * RPT, RPT)])
    plsc.subcore_barrier()

    npair = SPC // 2
    for s in range(NSTG):
        sbase = base + s * SPC
        pltpu.sync_copy(srcs_hbm.at[pl.ds(sbase, SPC)], srcb_v)
        pltpu.sync_copy(dsts_hbm.at[pl.ds(sbase, SPC)], dstb_v)
        pltpu.async_copy(xs_hbm.at[srcb_v.at[0]], rows0_v, sem0)
        _regcopy_row(dcur0_v, dstb_v, 0)

        def body(i, carry):
            a = 2 * i
            pltpu.async_copy(xs_hbm.at[srcb_v.at[a + 1]], rows1_v, sem1)
            _regcopy_row(dcur1_v, dstb_v, a + 1)
            pltpu.make_async_copy(xs_hbm, rows0_v, sem0).wait()
            pltpu.sync_copy(rows0_v, acc_s.at[dcur0_v], add=True)

            @pl.when(i < npair - 1)
            def _():
                pltpu.async_copy(xs_hbm.at[srcb_v.at[a + 2]], rows0_v, sem0)
                _regcopy_row(dcur0_v, dstb_v, a + 2)

            pltpu.make_async_copy(xs_hbm, rows1_v, sem1).wait()
            pltpu.sync_copy(rows1_v, acc_s.at[dcur1_v], add=True)
            return carry

        lax.fori_loop(0, npair, body, 0)
    plsc.subcore_barrier()
    pltpu.sync_copy(acc_s.at[pl.ds(sid * RPT, RPT)],
                    g_hbm.at[pl.ds(cid * NP + sid * RPT, RPT)])


def _deg_call():
    return pl.kernel(
        _deg_body,
        out_type=jax.ShapeDtypeStruct((4 * NP,), jnp.float32),
        mesh=_sc_mesh(),
        scratch_types=[
            pltpu.VMEM((CPT, C), jnp.int32),
            pltpu.VMEM((CPT, C), jnp.int32),
            pltpu.VMEM((C,), jnp.int32),
            pltpu.VMEM((C,), jnp.int32),
            pltpu.VMEM((C,), jnp.float32),
            pltpu.VMEM_SHARED((2 * NP,), jnp.float32),
            pltpu.VMEM_SHARED((NP,), jnp.float32),
        ],
    )


def _spmm_call():
    return pl.kernel(
        _spmm_body,
        out_type=jax.ShapeDtypeStruct((2 * NP, D), jnp.float32),
        mesh=_sc_mesh(),
        scratch_types=[
            pltpu.VMEM((SPC, C), jnp.int32),
            pltpu.VMEM((SPC, C), jnp.int32),
            pltpu.VMEM((C,), jnp.int32),
            pltpu.VMEM((C,), jnp.int32),
            pltpu.VMEM((C, D), jnp.float32),
            pltpu.VMEM((C, D), jnp.float32),
            pltpu.VMEM_SHARED((NP, D), jnp.float32),
            pltpu.SemaphoreType.DMA,
            pltpu.SemaphoreType.DMA,
        ],
    )


def _prep_body(xp_ref, degc_ref, xs_ref, csd_ref):
    iot = lax.broadcasted_iota(jnp.int32, (NP, 1), 0)
    dg = degc_ref[...]
    valid = (dg > 0) & (iot < N)
    csd = jnp.where(valid, lax.rsqrt(jnp.maximum(dg, 1.0)), 0.0)
    csd_ref[...] = csd
    xpv = xp_ref[...]
    xs_ref[0:NP, :] = xpv * csd[:, 0:1]
    xs_ref[NP:2 * NP, :] = xpv * csd[:, 2:3]


def _dense_mid_body(g_ref, csd_ref, w_ref, b_ref, xsn_ref):
    a0 = g_ref[0:N, :] * csd_ref[0:N, 1:2]
    a1 = g_ref[NP:NP + N, :] * csd_ref[0:N, 3:4]
    t = (jnp.dot(a0, w_ref[0], preferred_element_type=jnp.float32)
         + jnp.dot(a1, w_ref[1], preferred_element_type=jnp.float32)
         + b_ref[0:1, :] + b_ref[1:2, :])
    h = jnp.where(t > 0, t, 0.25 * t)
    xsn_ref[0:N, :] = h * csd_ref[0:N, 0:1]
    xsn_ref[N:NP, :] = jnp.zeros((NP - N, D), jnp.float32)
    xsn_ref[NP:NP + N, :] = h * csd_ref[0:N, 2:3]
    xsn_ref[NP + N:2 * NP, :] = jnp.zeros((NP - N, D), jnp.float32)


def _dense_last_body(g_ref, csd_ref, w_ref, b_ref, h_ref):
    a0 = g_ref[0:N, :] * csd_ref[0:N, 1:2]
    a1 = g_ref[NP:NP + N, :] * csd_ref[0:N, 3:4]
    t = (jnp.dot(a0, w_ref[0], preferred_element_type=jnp.float32)
         + jnp.dot(a1, w_ref[1], preferred_element_type=jnp.float32)
         + b_ref[0:1, :] + b_ref[1:2, :])
    h_ref[...] = jnp.where(t > 0, t, 0.25 * t)


_prep_call = pl.pallas_call(
    _prep_body,
    out_shape=(jax.ShapeDtypeStruct((2 * NP, D), jnp.float32),
               jax.ShapeDtypeStruct((NP, 4), jnp.float32)),
)

_dense_mid_call = pl.pallas_call(
    _dense_mid_body,
    out_shape=jax.ShapeDtypeStruct((2 * NP, D), jnp.float32),
)

_dense_last_call = pl.pallas_call(
    _dense_last_body,
    out_shape=jax.ShapeDtypeStruct((N, D), jnp.float32),
)


def kernel(x, edge_index_r0, edge_index_r1, W1, b1, W2, b2, W3, b3):
    srcs = jnp.stack([edge_index_r0[0], edge_index_r1[0]]).astype(jnp.int32)
    dsts = jnp.stack([edge_index_r0[1], edge_index_r1[1]]).astype(jnp.int32)
    srcs_p = jnp.pad(srcs, ((0, 0), (0, EP - E)), constant_values=N)
    dsts_p = jnp.pad(dsts, ((0, 0), (0, EP - E)), constant_values=N)
    srcs_g = (srcs_p + jnp.array([[0], [NP]], jnp.int32)).reshape(2 * NCH_R, C)
    dsts_g = dsts_p.reshape(2 * NCH_R, C)

    zflat = jnp.zeros((2 * NP,), jnp.float32)
    zrows = jnp.zeros((NP, D), jnp.float32)
    xp = jnp.pad(x.astype(jnp.float32), ((0, NP - N), (0, 0)))

    degs_flat = _deg_call()(srcs_g, dsts_g, zflat)
    degc = degs_flat.reshape(4, NP).T
    xs, csd = _prep_call(xp, degc)

    for li, (W, b) in enumerate(((W1, b1), (W2, b2), (W3, b3))):
        g = _spmm_call()(xs, srcs_g, dsts_g, zrows)
        if li < 2:
            xs = _dense_mid_call(g, csd, W, b)
        else:
            return _dense_last_call(g, csd, W, b)
